# baseline (device time: 111593 ns/iter reference)
import jax
import jax.numpy as jnp
from jax import lax
from jax.experimental import pallas as pl
from jax.experimental.pallas import tpu as pltpu

N_DEV = 4
B_SH = 64
B = 256
D = 2048
H_SH = 4096
KT_IN = 512
KT_HID = 1024
C_HALF = D // 2
N_TI = D // KT_IN
N_TO = H_SH // KT_HID


def kernel(x, Win0, Wout0, Win1, Wout1, Win2, Wout2):
    def body(x_ref, win0, wout0, win1, wout1, win2, wout2, out_ref,
             x_full, pbuf, rsrecv, winbuf, winbf, woutbuf, woutbf,
             winsem, woutsem, rs_send, rs_recv, ag_send, ag_recv):
        my = lax.axis_index("i")
        left = (my - 1) % N_DEV
        right = (my + 1) % N_DEV

        wins = (win0, win1, win2)
        wouts = (wout0, wout1, wout2)

        win_jobs = [(l, t) for l in range(3) for t in range(N_TI)]
        wout_jobs = [(l, c, t) for l in range(3) for c in range(2)
                     for t in range(N_TO)]
        win_descs = {}
        wout_descs = {}

        def start_win(k):
            if k < len(win_jobs):
                l, t = win_jobs[k]
                cp = pltpu.make_async_copy(
                    wins[l].at[pl.ds(t * KT_IN, KT_IN), :],
                    winbuf.at[k % 2], winsem.at[k % 2])
                cp.start()
                win_descs[k] = cp

        def start_wout(k):
            if k < len(wout_jobs):
                l, c, t = wout_jobs[k]
                cp = pltpu.make_async_copy(
                    wouts[l].at[pl.ds(t * KT_HID, KT_HID),
                                pl.ds(c * C_HALF, C_HALF)],
                    woutbuf.at[k % 2], woutsem.at[k % 2])
                cp.start()
                wout_descs[k] = cp

        def cast_win(k):
            if k < len(win_jobs):
                win_descs[k].wait()
                winbf[k % 2] = winbuf[k % 2].astype(jnp.bfloat16)
                start_win(k + 2)

        def cast_wout(k):
            if k < len(wout_jobs):
                wout_descs[k].wait()
                woutbf[k % 2] = woutbuf[k % 2].astype(jnp.bfloat16)
                start_wout(k + 2)

        start_win(0)
        start_win(1)
        start_wout(0)
        start_wout(1)

        barrier = pltpu.get_barrier_semaphore()
        for nbr in (left, right):
            pl.semaphore_signal(barrier, inc=1, device_id=(nbr,),
                                device_id_type=pl.DeviceIdType.MESH)
        pl.semaphore_wait(barrier, 2)

        def ag_start_half(c):
            csl = pl.ds(c * C_HALF, C_HALF)
            rdmas = []
            for j in (1, 2, 3):
                dst = (my + j) % N_DEV
                s = (j - 1) * 2 + c
                r = pltpu.make_async_remote_copy(
                    src_ref=x_full.at[pl.ds(0, B_SH), csl],
                    dst_ref=x_full.at[pl.ds((N_DEV - j) * B_SH, B_SH), csl],
                    send_sem=ag_send.at[s], recv_sem=ag_recv.at[s],
                    device_id=(dst,), device_id_type=pl.DeviceIdType.MESH)
                r.start()
                rdmas.append(r)
            return rdmas

        def rs_start_half(c):
            csl = pl.ds(c * C_HALF, C_HALF)
            rdmas = []
            for j in (1, 2, 3):
                dst = (my + j) % N_DEV
                s = (j - 1) * 2 + c
                r = pltpu.make_async_remote_copy(
                    src_ref=pbuf.at[pl.ds(j * B_SH, B_SH), csl],
                    dst_ref=rsrecv.at[j - 1, slice(None), csl],
                    send_sem=rs_send.at[s], recv_sem=rs_recv.at[s],
                    device_id=(dst,), device_id_type=pl.DeviceIdType.MESH)
                r.start()
                rdmas.append(r)
            return rdmas

        def reduce_half(c, rdmas):
            for r in rdmas:
                r.wait_recv()
            lo, hi = c * C_HALF, (c + 1) * C_HALF
            red = pbuf[0:B_SH, lo:hi].astype(jnp.float32)
            for kk in range(3):
                red = red + rsrecv[kk, :, lo:hi].astype(jnp.float32)
            x_full[0:B_SH, lo:hi] = red.astype(jnp.bfloat16)
            return ag_start_half(c)

        x_full[0:B_SH, :] = x_ref[...].astype(jnp.bfloat16)
        ag = ag_start_half(0) + ag_start_half(1)
        cast_win(0)
        cast_win(1)
        cast_wout(0)
        cast_wout(1)
        for r in ag:
            r.wait_recv()
        for r in ag:
            r.wait_send()

        for l in range(3):
            hacc = None
            for t in range(N_TI):
                k = l * N_TI + t
                part = jnp.dot(x_full[:, t * KT_IN:(t + 1) * KT_IN],
                               winbf[k % 2],
                               preferred_element_type=jnp.float32)
                hacc = part if hacc is None else hacc + part
                if t < N_TI - 2:
                    cast_win(k + 2)
            hb = jnp.maximum(hacc, 0.0).astype(jnp.bfloat16)

            kb = l * 2 * N_TO
            pacc = None
            for t in range(N_TO):
                k = kb + t
                p = jnp.dot(hb[:, t * KT_HID:(t + 1) * KT_HID], woutbf[k % 2],
                            preferred_element_type=jnp.float32)
                pacc = p if pacc is None else pacc + p
                cast_wout(k + 2)
            pbuf[:, 0:C_HALF] = pacc.astype(jnp.bfloat16)
            rs0 = rs_start_half(0)

            pacc = None
            ag_all = None
            for t in range(N_TO):
                k = kb + N_TO + t
                p = jnp.dot(hb[:, t * KT_HID:(t + 1) * KT_HID], woutbf[k % 2],
                            preferred_element_type=jnp.float32)
                pacc = p if pacc is None else pacc + p
                if t < N_TO - 2:
                    cast_wout(k + 2)
                if t == 1:
                    ag_all = reduce_half(0, rs0)
            pbuf[:, C_HALF:D] = pacc.astype(jnp.bfloat16)
            rs1 = rs_start_half(1)

            if l + 1 < 3:
                cast_win((l + 1) * N_TI)
                cast_win((l + 1) * N_TI + 1)
            ag_all += reduce_half(1, rs1)
            if l + 1 < 3:
                cast_wout((l + 1) * 2 * N_TO)
                cast_wout((l + 1) * 2 * N_TO + 1)
            if l + 1 == 3:
                out_ref[pl.ds(my * B_SH, B_SH), :] = (
                    x_full[0:B_SH, :].astype(jnp.float32))
            for r in ag_all:
                r.wait_recv()
            for r in ag_all:
                r.wait_send()
            for r in rs0 + rs1:
                r.wait_send()

        for j in range(1, N_DEV):
            gb = (my + j) % N_DEV
            out_ref[pl.ds(gb * B_SH, B_SH), :] = (
                x_full[j * B_SH:(j + 1) * B_SH, :].astype(jnp.float32))

    return pl.pallas_call(
        body,
        out_shape=jax.ShapeDtypeStruct((B, D), jnp.float32),
        in_specs=[pl.BlockSpec(memory_space=pltpu.VMEM)]
        + [pl.BlockSpec(memory_space=pl.ANY)] * 6,
        out_specs=pl.BlockSpec(memory_space=pltpu.VMEM),
        scratch_shapes=[
            pltpu.VMEM((B, D), jnp.bfloat16),
            pltpu.VMEM((B, D), jnp.bfloat16),
            pltpu.VMEM((3, B_SH, D), jnp.bfloat16),
            pltpu.VMEM((2, KT_IN, H_SH), jnp.float32),
            pltpu.VMEM((2, KT_IN, H_SH), jnp.bfloat16),
            pltpu.VMEM((2, KT_HID, C_HALF), jnp.float32),
            pltpu.VMEM((2, KT_HID, C_HALF), jnp.bfloat16),
            pltpu.SemaphoreType.DMA((2,)),
            pltpu.SemaphoreType.DMA((2,)),
            pltpu.SemaphoreType.DMA((6,)),
            pltpu.SemaphoreType.DMA((6,)),
            pltpu.SemaphoreType.DMA((6,)),
            pltpu.SemaphoreType.DMA((6,)),
        ],
        compiler_params=pltpu.CompilerParams(
            collective_id=0, vmem_limit_bytes=62 * 1024 * 1024),
    )(x, Win0, Wout0, Win1, Wout1, Win2, Wout2)


# device time: 102292 ns/iter; 1.0909x vs baseline; 1.0909x over previous
import jax
import jax.numpy as jnp
from jax import lax
from jax.experimental import pallas as pl
from jax.experimental.pallas import tpu as pltpu

N_DEV = 4
B_SH = 64
B = 256
D = 2048
H_SH = 4096
KT_IN = 512
KT_HID = 1024
C_HALF = D // 2
N_TI = D // KT_IN
N_TO = H_SH // KT_HID


def kernel(x, Win0, Wout0, Win1, Wout1, Win2, Wout2):
    def body(x_ref, win0, wout0, win1, wout1, win2, wout2, out_ref,
             x_full, pbuf, rsrecv, hb, winbuf, winbf, woutbuf, woutbf,
             winsem, woutsem, rs_send, rs_recv, ag_send, ag_recv):
        my = lax.axis_index("i")
        left = (my - 1) % N_DEV
        right = (my + 1) % N_DEV

        wins = (win0, win1, win2)
        wouts = (wout0, wout1, wout2)

        win_jobs = [(l, t) for l in range(3) for t in range(N_TI)]
        wout_jobs = [(l, c, t) for l in range(3) for c in range(2)
                     for t in range(N_TO)]
        win_descs = {}
        wout_descs = {}

        def start_win(k):
            if k < len(win_jobs):
                l, t = win_jobs[k]
                cp = pltpu.make_async_copy(
                    wins[l].at[pl.ds(t * KT_IN, KT_IN), :],
                    winbuf.at[k % 2], winsem.at[k % 2])
                cp.start()
                win_descs[k] = cp

        def start_wout(k):
            if k < len(wout_jobs):
                l, c, t = wout_jobs[k]
                cp = pltpu.make_async_copy(
                    wouts[l].at[pl.ds(t * KT_HID, KT_HID),
                                pl.ds(c * C_HALF, C_HALF)],
                    woutbuf.at[k % 2], woutsem.at[k % 2])
                cp.start()
                wout_descs[k] = cp

        def cast_win(k):
            if k < len(win_jobs):
                win_descs[k].wait()
                winbf[k % 4] = winbuf[k % 2].astype(jnp.bfloat16)
                start_win(k + 2)

        def cast_wout(k):
            if k < len(wout_jobs):
                wout_descs[k].wait()
                woutbf[k % 2] = woutbuf[k % 2].astype(jnp.bfloat16)
                start_wout(k + 2)

        start_win(0)
        start_win(1)
        start_wout(0)
        start_wout(1)

        barrier = pltpu.get_barrier_semaphore()
        for nbr in (left, right):
            pl.semaphore_signal(barrier, inc=1, device_id=(nbr,),
                                device_id_type=pl.DeviceIdType.MESH)
        pl.semaphore_wait(barrier, 2)

        def ag_start_half(c):
            csl = pl.ds(c * C_HALF, C_HALF)
            rdmas = []
            for j in (1, 2, 3):
                dst = (my + j) % N_DEV
                s = (j - 1) * 2 + c
                r = pltpu.make_async_remote_copy(
                    src_ref=x_full.at[pl.ds(0, B_SH), csl],
                    dst_ref=x_full.at[pl.ds((N_DEV - j) * B_SH, B_SH), csl],
                    send_sem=ag_send.at[s], recv_sem=ag_recv.at[s],
                    device_id=(dst,), device_id_type=pl.DeviceIdType.MESH)
                r.start()
                rdmas.append(r)
            return rdmas

        def rs_start_half(c):
            csl = pl.ds(c * C_HALF, C_HALF)
            rdmas = []
            for j in (1, 2, 3):
                dst = (my + j) % N_DEV
                s = (j - 1) * 2 + c
                r = pltpu.make_async_remote_copy(
                    src_ref=pbuf.at[pl.ds(j * B_SH, B_SH), csl],
                    dst_ref=rsrecv.at[j - 1, slice(None), csl],
                    send_sem=rs_send.at[s], recv_sem=rs_recv.at[s],
                    device_id=(dst,), device_id_type=pl.DeviceIdType.MESH)
                r.start()
                rdmas.append(r)
            return rdmas

        def block0_dots(l, ts, hacc0):
            for t in ts:
                part = jnp.dot(x_full[0:B_SH, t * KT_IN:(t + 1) * KT_IN],
                               winbf[(l * N_TI + t) % 4],
                               preferred_element_type=jnp.float32)
                hacc0 = part if hacc0 is None else hacc0 + part
            return hacc0

        x_full[0:B_SH, :] = x_ref[...].astype(jnp.bfloat16)
        ag = ag_start_half(0) + ag_start_half(1)
        cast_win(0)
        cast_win(1)
        cast_wout(0)
        cast_wout(1)
        hacc0 = block0_dots(0, (0, 1), None)
        cast_win(2)
        cast_win(3)
        hacc0 = block0_dots(0, (2, 3), hacc0)
        hb[0:B_SH, :] = jnp.maximum(hacc0, 0.0).astype(jnp.bfloat16)
        for r in ag:
            r.wait_recv()
        for r in ag:
            r.wait_send()

        for l in range(3):
            haccr = None
            for t in range(N_TI):
                part = jnp.dot(
                    x_full[B_SH:B, t * KT_IN:(t + 1) * KT_IN],
                    winbf[(l * N_TI + t) % 4],
                    preferred_element_type=jnp.float32)
                haccr = part if haccr is None else haccr + part
            hb[B_SH:B, :] = jnp.maximum(haccr, 0.0).astype(jnp.bfloat16)

            kb = l * 2 * N_TO
            rs_all = []
            for c in range(2):
                pacc = None
                for t in range(N_TO):
                    k = kb + c * N_TO + t
                    p = jnp.dot(hb[:, t * KT_HID:(t + 1) * KT_HID],
                                woutbf[k % 2],
                                preferred_element_type=jnp.float32)
                    pacc = p if pacc is None else pacc + p
                    if not (c == 1 and t >= N_TO - 2):
                        cast_wout(k + 2)
                pbuf[:, c * C_HALF:(c + 1) * C_HALF] = pacc.astype(jnp.bfloat16)
                rs_all.append(rs_start_half(c))

            last = l + 1 == 3
            nw = (l + 1) * N_TI
            if not last:
                cast_win(nw)
                cast_win(nw + 1)
            for r in rs_all[0]:
                r.wait_recv()
            c0 = pl.ds(0, C_HALF)
            red = pbuf[0:B_SH, 0:C_HALF].astype(jnp.float32)
            for kk in range(3):
                red = red + rsrecv[kk, :, 0:C_HALF].astype(jnp.float32)
            x_full[0:B_SH, 0:C_HALF] = red.astype(jnp.bfloat16)
            ag_all = ag_start_half(0)
            hacc0 = None
            if not last:
                hacc0 = block0_dots(l + 1, (0, 1), None)
                cast_wout((l + 1) * 2 * N_TO)
                cast_wout((l + 1) * 2 * N_TO + 1)
            for r in rs_all[1]:
                r.wait_recv()
            red = pbuf[0:B_SH, C_HALF:D].astype(jnp.float32)
            for kk in range(3):
                red = red + rsrecv[kk, :, C_HALF:D].astype(jnp.float32)
            x_full[0:B_SH, C_HALF:D] = red.astype(jnp.bfloat16)
            ag_all += ag_start_half(1)
            if not last:
                cast_win(nw + 2)
                cast_win(nw + 3)
                hacc0 = block0_dots(l + 1, (2, 3), hacc0)
                hb[0:B_SH, :] = jnp.maximum(hacc0, 0.0).astype(jnp.bfloat16)
            for r in ag_all:
                r.wait_recv()
            for r in ag_all:
                r.wait_send()
            for rl in rs_all:
                for r in rl:
                    r.wait_send()

        for j in range(N_DEV):
            gb = (my + j) % N_DEV
            out_ref[pl.ds(gb * B_SH, B_SH), :] = (
                x_full[j * B_SH:(j + 1) * B_SH, :].astype(jnp.float32))

    return pl.pallas_call(
        body,
        out_shape=jax.ShapeDtypeStruct((B, D), jnp.float32),
        in_specs=[pl.BlockSpec(memory_space=pltpu.VMEM)]
        + [pl.BlockSpec(memory_space=pl.ANY)] * 6,
        out_specs=pl.BlockSpec(memory_space=pltpu.VMEM),
        scratch_shapes=[
            pltpu.VMEM((B, D), jnp.bfloat16),
            pltpu.VMEM((B, D), jnp.bfloat16),
            pltpu.VMEM((3, B_SH, D), jnp.bfloat16),
            pltpu.VMEM((B, H_SH), jnp.bfloat16),
            pltpu.VMEM((2, KT_IN, H_SH), jnp.float32),
            pltpu.VMEM((4, KT_IN, H_SH), jnp.bfloat16),
            pltpu.VMEM((2, KT_HID, C_HALF), jnp.float32),
            pltpu.VMEM((2, KT_HID, C_HALF), jnp.bfloat16),
            pltpu.SemaphoreType.DMA((2,)),
            pltpu.SemaphoreType.DMA((2,)),
            pltpu.SemaphoreType.DMA((6,)),
            pltpu.SemaphoreType.DMA((6,)),
            pltpu.SemaphoreType.DMA((6,)),
            pltpu.SemaphoreType.DMA((6,)),
        ],
        compiler_params=pltpu.CompilerParams(
            collective_id=0, vmem_limit_bytes=62 * 1024 * 1024),
    )(x, Win0, Wout0, Win1, Wout1, Win2, Wout2)
